# baseline (speedup 1.0000x reference)
import functools

import jax
import jax.numpy as jnp
from jax.experimental import pallas as pl
from jax.experimental.pallas import tpu as pltpu

HEADS = 4
GH = 64
NB = 2048
N_DIM = 22


def _gelu_exact(x):
    return 0.5 * x * (1.0 + jax.lax.erf(x * 0.7071067811865476))


def _ln(x, g, b, eps=1e-5):
    mu = jnp.mean(x, axis=-1, keepdims=True)
    var = jnp.mean((x - mu) ** 2, axis=-1, keepdims=True)
    return (x - mu) / jnp.sqrt(var + eps) * g + b


def _add_loops(edge_index, edge_attr, n):
    src, dst = edge_index[0], edge_index[1]
    ones = jnp.ones((src.shape[0],), dtype=jnp.float32)
    cnt = jax.ops.segment_sum(ones, dst, num_segments=n)
    loop_attr = jax.ops.segment_sum(edge_attr, dst, num_segments=n) / jnp.maximum(cnt, 1.0)[:, None]
    loop = jnp.arange(n, dtype=src.dtype)
    return (jnp.concatenate([src, loop]), jnp.concatenate([dst, loop]),
            jnp.concatenate([edge_attr, loop_attr], axis=0))


def _gat_layer(x, src, dst, ea, W, aS, aD, We, aE, bias, heads, ch, concat):
    n = x.shape[0]
    h = (x @ W).reshape(n, heads, ch)
    e = (ea @ We).reshape(-1, heads, ch)
    a_src = jnp.sum(h * aS[None], axis=-1)
    a_dst = jnp.sum(h * aD[None], axis=-1)
    a_e = jnp.sum(e * aE[None], axis=-1)
    alpha = jax.nn.leaky_relu(a_src[src] + a_dst[dst] + a_e, negative_slope=0.2)
    amax = jax.ops.segment_max(alpha, dst, num_segments=n)
    amax = jnp.where(jnp.isfinite(amax), amax, 0.0)
    ex = jnp.exp(alpha - amax[dst])
    denom = jax.ops.segment_sum(ex, dst, num_segments=n)
    coef = ex / (denom[dst] + 1e-16)
    out = jax.ops.segment_sum(h[src] * coef[:, :, None], dst, num_segments=n)
    if concat:
        out = out.reshape(n, heads * ch)
    else:
        out = out.mean(axis=1)
    return out + bias


def _head_body(graph_ref, bert_ref, bpW_ref, bpb_ref, bplng_ref, bplnb_ref,
               f1Wg_ref, f1Wb_ref, f1b_ref, fln1g_ref, fln1b_ref,
               f2W_ref, f2b_ref, fln2g_ref, fln2b_ref, f3W_ref, f3b_ref,
               out_ref):
    bert = bert_ref[...]
    bp = _gelu_exact(bert @ bpW_ref[...] + bpb_ref[...])
    bert_feat = _ln(bp, bplng_ref[...], bplnb_ref[...])
    g = graph_ref[...]
    z = g @ f1Wg_ref[...] + bert_feat @ f1Wb_ref[...] + f1b_ref[...]
    z = _ln(_gelu_exact(z), fln1g_ref[...], fln1b_ref[...])
    z = _ln(_gelu_exact(z @ f2W_ref[...] + f2b_ref[...]),
            fln2g_ref[...], fln2b_ref[...])
    out_ref[...] = jax.nn.sigmoid(z @ f3W_ref[...] + f3b_ref[...])


def _head(graph_feat, bert, bpW, bpb, bplng, bplnb, f1W, f1b, fln1g, fln1b,
          f2W, f2b, fln2g, fln2b, f3W, f3b, interpret=False):
    Bn = graph_feat.shape[0]
    BLK = 256
    f1Wg = f1W[:128]
    f1Wb = f1W[128:]
    row = lambda i: (i, 0)
    full = lambda s: pl.BlockSpec(s, lambda i: (0, 0))
    full1 = lambda s: pl.BlockSpec(s, lambda i: (0,))
    return pl.pallas_call(
        _head_body,
        grid=(Bn // BLK,),
        in_specs=[
            pl.BlockSpec((BLK, 128), row),
            pl.BlockSpec((BLK, 384), row),
            full((384, 256)), full1((256,)), full1((256,)), full1((256,)),
            full((128, 256)), full((256, 256)), full1((256,)), full1((256,)), full1((256,)),
            full((256, 128)), full1((128,)), full1((128,)), full1((128,)),
            full((128, N_DIM)), full1((N_DIM,)),
        ],
        out_specs=pl.BlockSpec((BLK, N_DIM), row),
        out_shape=jax.ShapeDtypeStruct((Bn, N_DIM), jnp.float32),
        interpret=interpret,
    )(graph_feat, bert, bpW, bpb, bplng, bplnb, f1Wg, f1Wb, f1b, fln1g, fln1b,
      f2W, f2b, fln2g, fln2b, f3W, f3b)


def kernel(x, edge_index, edge_attr, batch, bert_embedding, W1, aS1, aD1, We1, aE1, b1, ln1g, ln1b, W2, aS2, aD2, We2, aE2, b2, ln2g, ln2b, W3, aS3, aD3, We3, aE3, b3, ln3g, ln3b, bpW, bpb, bplng, bplnb, f1W, f1b, fln1g, fln1b, f2W, f2b, fln2g, fln2b, f3W, f3b):
    n = x.shape[0]
    src, dst, ea = _add_loops(edge_index, edge_attr, n)
    h = jax.nn.elu(_ln(_gat_layer(x, src, dst, ea, W1, aS1, aD1, We1, aE1, b1, HEADS, GH, True), ln1g, ln1b))
    h = jax.nn.elu(_ln(_gat_layer(h, src, dst, ea, W2, aS2, aD2, We2, aE2, b2, HEADS, GH, True), ln2g, ln2b))
    h = jax.nn.elu(_ln(_gat_layer(h, src, dst, ea, W3, aS3, aD3, We3, aE3, b3, 1, GH, False), ln3g, ln3b))
    ones = jnp.ones((n,), dtype=jnp.float32)
    cnt = jax.ops.segment_sum(ones, batch, num_segments=NB)
    h_mean = jax.ops.segment_sum(h, batch, num_segments=NB) / jnp.maximum(cnt, 1.0)[:, None]
    h_max = jax.ops.segment_max(h, batch, num_segments=NB)
    h_max = jnp.where(jnp.isfinite(h_max), h_max, 0.0)
    graph_feat = jnp.concatenate([h_mean, h_max], axis=-1)
    return _head(graph_feat, bert_embedding, bpW, bpb, bplng, bplnb,
                 f1W, f1b, fln1g, fln1b, f2W, f2b, fln2g, fln2b, f3W, f3b)


# speedup vs baseline: 1.0044x; 1.0044x over previous
import functools

import jax
import jax.numpy as jnp
from jax.experimental import pallas as pl
from jax.experimental.pallas import tpu as pltpu

HEADS = 4
GH = 64
NB = 2048
N_DIM = 22


def _gelu_exact(x):
    return 0.5 * x * (1.0 + jax.lax.erf(x * 0.7071067811865476))


def _ln(x, g, b, eps=1e-5):
    mu = jnp.mean(x, axis=-1, keepdims=True)
    var = jnp.mean((x - mu) ** 2, axis=-1, keepdims=True)
    return (x - mu) / jnp.sqrt(var + eps) * g + b


def _add_loops(edge_index, edge_attr, n):
    src, dst = edge_index[0], edge_index[1]
    ones = jnp.ones((src.shape[0],), dtype=jnp.float32)
    cnt = jax.ops.segment_sum(ones, dst, num_segments=n)
    loop_attr = jax.ops.segment_sum(edge_attr, dst, num_segments=n) / jnp.maximum(cnt, 1.0)[:, None]
    loop = jnp.arange(n, dtype=src.dtype)
    return (jnp.concatenate([src, loop]), jnp.concatenate([dst, loop]),
            jnp.concatenate([edge_attr, loop_attr], axis=0))


def _ae_body(ea_ref, w_ref, out_ref):
    out_ref[...] = ea_ref[...] @ w_ref[0:3, :]


def _ae_prep(ea, wcat):
    EB = 2000
    ne = ea.shape[0]
    return pl.pallas_call(
        _ae_body,
        grid=(ne // EB,),
        in_specs=[
            pl.BlockSpec((EB, 3), lambda i: (i, 0)),
            pl.BlockSpec((8, 16), lambda i: (0, 0)),
        ],
        out_specs=pl.BlockSpec((EB, 16), lambda i: (i, 0)),
        out_shape=jax.ShapeDtypeStruct((ne, 16), jnp.float32),
    )(ea, wcat)


def _gat_layer(x, src, dst, ea, W, aS, aD, a_e, bias, heads, ch, concat):
    n = x.shape[0]
    h = (x @ W).reshape(n, heads, ch)
    a_src = jnp.sum(h * aS[None], axis=-1)
    a_dst = jnp.sum(h * aD[None], axis=-1)
    alpha = jax.nn.leaky_relu(a_src[src] + a_dst[dst] + a_e, negative_slope=0.2)
    amax = jax.ops.segment_max(alpha, dst, num_segments=n)
    amax = jnp.where(jnp.isfinite(amax), amax, 0.0)
    ex = jnp.exp(alpha - amax[dst])
    denom = jax.ops.segment_sum(ex, dst, num_segments=n)
    coef = ex / (denom[dst] + 1e-16)
    out = jax.ops.segment_sum(h[src] * coef[:, :, None], dst, num_segments=n)
    if concat:
        out = out.reshape(n, heads * ch)
    else:
        out = out.mean(axis=1)
    return out + bias


def _head_body(graph_ref, bert_ref, bpW_ref, bpb_ref, bplng_ref, bplnb_ref,
               f1Wg_ref, f1Wb_ref, f1b_ref, fln1g_ref, fln1b_ref,
               f2W_ref, f2b_ref, fln2g_ref, fln2b_ref, f3W_ref, f3b_ref,
               out_ref):
    bert = bert_ref[...]
    bp = _gelu_exact(bert @ bpW_ref[...] + bpb_ref[...])
    bert_feat = _ln(bp, bplng_ref[...], bplnb_ref[...])
    g = graph_ref[...]
    z = g @ f1Wg_ref[...] + bert_feat @ f1Wb_ref[...] + f1b_ref[...]
    z = _ln(_gelu_exact(z), fln1g_ref[...], fln1b_ref[...])
    z = _ln(_gelu_exact(z @ f2W_ref[...] + f2b_ref[...]),
            fln2g_ref[...], fln2b_ref[...])
    out_ref[...] = jax.nn.sigmoid(z @ f3W_ref[...] + f3b_ref[...])


def _head(graph_feat, bert, bpW, bpb, bplng, bplnb, f1W, f1b, fln1g, fln1b,
          f2W, f2b, fln2g, fln2b, f3W, f3b, interpret=False):
    Bn = graph_feat.shape[0]
    BLK = 256
    f1Wg = f1W[:128]
    f1Wb = f1W[128:]
    row = lambda i: (i, 0)
    full = lambda s: pl.BlockSpec(s, lambda i: (0, 0))
    full1 = lambda s: pl.BlockSpec(s, lambda i: (0,))
    return pl.pallas_call(
        _head_body,
        grid=(Bn // BLK,),
        in_specs=[
            pl.BlockSpec((BLK, 128), row),
            pl.BlockSpec((BLK, 384), row),
            full((384, 256)), full1((256,)), full1((256,)), full1((256,)),
            full((128, 256)), full((256, 256)), full1((256,)), full1((256,)), full1((256,)),
            full((256, 128)), full1((128,)), full1((128,)), full1((128,)),
            full((128, N_DIM)), full1((N_DIM,)),
        ],
        out_specs=pl.BlockSpec((BLK, N_DIM), row),
        out_shape=jax.ShapeDtypeStruct((Bn, N_DIM), jnp.float32),
        interpret=interpret,
    )(graph_feat, bert, bpW, bpb, bplng, bplnb, f1Wg, f1Wb, f1b, fln1g, fln1b,
      f2W, f2b, fln2g, fln2b, f3W, f3b)


def kernel(x, edge_index, edge_attr, batch, bert_embedding, W1, aS1, aD1, We1, aE1, b1, ln1g, ln1b, W2, aS2, aD2, We2, aE2, b2, ln2g, ln2b, W3, aS3, aD3, We3, aE3, b3, ln3g, ln3b, bpW, bpb, bplng, bplnb, f1W, f1b, fln1g, fln1b, f2W, f2b, fln2g, fln2b, f3W, f3b):
    n = x.shape[0]
    src, dst, ea = _add_loops(edge_index, edge_attr, n)

    def fold(We, aE, heads):
        return (We.reshape(3, heads, GH) * aE[None]).sum(-1)

    wcat = jnp.zeros((8, 16), jnp.float32)
    wcat = wcat.at[0:3, 0:4].set(fold(We1, aE1, HEADS))
    wcat = wcat.at[0:3, 4:8].set(fold(We2, aE2, HEADS))
    wcat = wcat.at[0:3, 8:9].set(fold(We3, aE3, 1))
    nea = ea.shape[0]
    pad = (-nea) % 2000
    ae_all = _ae_prep(jnp.concatenate(
        [ea, jnp.zeros((pad, 3), ea.dtype)], axis=0), wcat)[:nea]
    h = jax.nn.elu(_ln(_gat_layer(x, src, dst, ea, W1, aS1, aD1, ae_all[:, 0:4], b1, HEADS, GH, True), ln1g, ln1b))
    h = jax.nn.elu(_ln(_gat_layer(h, src, dst, ea, W2, aS2, aD2, ae_all[:, 4:8], b2, HEADS, GH, True), ln2g, ln2b))
    h = jax.nn.elu(_ln(_gat_layer(h, src, dst, ea, W3, aS3, aD3, ae_all[:, 8:9], b3, 1, GH, False), ln3g, ln3b))
    ones = jnp.ones((n,), dtype=jnp.float32)
    cnt = jax.ops.segment_sum(ones, batch, num_segments=NB)
    h_mean = jax.ops.segment_sum(h, batch, num_segments=NB) / jnp.maximum(cnt, 1.0)[:, None]
    h_max = jax.ops.segment_max(h, batch, num_segments=NB)
    h_max = jnp.where(jnp.isfinite(h_max), h_max, 0.0)
    graph_feat = jnp.concatenate([h_mean, h_max], axis=-1)
    return _head(graph_feat, bert_embedding, bpW, bpb, bplng, bplnb,
                 f1W, f1b, fln1g, fln1b, f2W, f2b, fln2g, fln2b, f3W, f3b)
